# Initial kernel scaffold; baseline (speedup 1.0000x reference)
import jax
import jax.numpy as jnp
from jax.experimental import pallas as pl


def kernel(x_user, x_item, edge_index_user_to_item, edge_index_item_to_user, params):
    raise NotImplementedError("write your pallas kernel here")


# speedup vs baseline: 1.3288x; 1.3288x over previous
import jax
import jax.numpy as jnp
from jax.experimental import pallas as pl
from jax.experimental.pallas import tpu as pltpu

D = 256
N = 10000
RSQRT_D = 1.0 / 16.0

NODE_TYPES = ("user", "item")
EDGE_TYPES = (("user", "item"), ("item", "user"))


def _mm_kernel(x_ref, w_ref, b_ref, o_ref):
    o_ref[...] = (
        jnp.dot(
            x_ref[...],
            w_ref[...],
            preferred_element_type=jnp.float32,
            precision=jax.lax.Precision.HIGHEST,
        )
        + b_ref[...]
    )


def _lin_pallas(x, w, b):
    n, d = x.shape
    o = w.shape[0]
    blk = 400
    return pl.pallas_call(
        _mm_kernel,
        grid=(n // blk,),
        in_specs=[
            pl.BlockSpec((blk, d), lambda i: (i, 0)),
            pl.BlockSpec((d, o), lambda i: (0, 0)),
            pl.BlockSpec((1, o), lambda i: (0, 0)),
        ],
        out_specs=pl.BlockSpec((blk, o), lambda i: (i, 0)),
        out_shape=jax.ShapeDtypeStruct((n, o), jnp.float32),
    )(x, w.T, b[None, :])


def _layer(xd, eid, lp):
    out = {}
    q = {}
    k = {}
    v = {}
    for nt in NODE_TYPES:
        x = xd[nt]
        q[nt] = _lin_pallas(x, lp["q"][nt]["w"], lp["q"][nt]["b"])
        k[nt] = _lin_pallas(x, lp["k"][nt]["w"], lp["k"][nt]["b"])
        v[nt] = _lin_pallas(x, lp["v"][nt]["w"], lp["v"][nt]["b"])

    for nt in NODE_TYPES:
        n = xd[nt].shape[0]
        ra = jnp.sum(q[nt] * k[nt], axis=-1) * RSQRT_D
        self_msg = jax.nn.softmax(ra, axis=0)[:, None] * v[nt]
        acc = jnp.zeros_like(xd[nt])
        for (st, dt) in EDGE_TYPES:
            if dt != nt:
                continue
            ekey = st + "__" + dt
            eidx = eid[ekey]
            src = eidx[0]
            dst = eidx[1]
            W = lp["W"][ekey]
            cnt = jax.ops.segment_sum(
                jnp.ones((dst.shape[0],), jnp.float32), dst, num_segments=n
            )
            kbar = jnp.sum(cnt[:, None] * k[dt], axis=0)
            p = ((q[st] @ W) @ kbar) * RSQRT_D
            raw = p[src]
            m = jax.ops.segment_max(raw, dst, num_segments=n)
            m = jnp.where(jnp.isfinite(m), m, 0.0)
            w_e = jnp.exp(raw - m[dst])
            s = jax.ops.segment_sum(w_e, dst, num_segments=n)
            numer = jax.ops.segment_sum(w_e[:, None] * v[st][src], dst, num_segments=n)
            deg = jnp.clip(cnt, 1.0, None)
            acc = acc + numer / (jnp.maximum(s, 0.5) * deg)[:, None]
            acc = acc + self_msg
        out[nt] = acc
    return out


def kernel(x_user, x_item, edge_index_user_to_item, edge_index_item_to_user, params):
    xd = {"user": x_user, "item": x_item}
    eid = {
        "user__item": edge_index_user_to_item,
        "item__user": edge_index_item_to_user,
    }
    xd = _layer(xd, eid, params["l1"])
    xd = _layer(xd, eid, params["l2"])
    return tuple(
        xd[nt] + _lin_pallas(xd[nt], params["self"][nt]["w"], params["self"][nt]["b"])
        for nt in NODE_TYPES
    )


# speedup vs baseline: 3.0993x; 2.3324x over previous
import functools

import jax
import jax.numpy as jnp
from jax import lax
from jax.experimental import pallas as pl
from jax.experimental.pallas import tpu as pltpu, tpu_sc as plsc

D = 256
N = 10000
E = 160000
NP = 10240
T = 320
GARB = 320
SEGW = 352
LCAP = 768
CH = 4000
NCH = E // CH
RSQRT_D = 1.0 / 16.0
BLK = 400
GRID = N // BLK

def _mesh():
    return plsc.VectorSubcoreMesh(
        core_axis_name="c", subcore_axis_name="s", num_cores=2, num_subcores=16
    )


def _wid():
    return lax.axis_index("c") * 16 + lax.axis_index("s")


def _lane():
    return lax.iota(jnp.int32, 16)


@functools.cache
def _build_preproc():
    return functools.partial(
        pl.kernel,
        out_type=(
            jax.ShapeDtypeStruct((32, 16 * LCAP), jnp.int32),
            jax.ShapeDtypeStruct((32, 16), jnp.int32),
            jax.ShapeDtypeStruct((NP,), jnp.float32),
            jax.ShapeDtypeStruct((32, 16 * LCAP), jnp.int32),
            jax.ShapeDtypeStruct((32, 16), jnp.int32),
            jax.ShapeDtypeStruct((NP,), jnp.float32),
        ),
        mesh=_mesh(),
        compiler_params=pltpu.CompilerParams(needs_layout_passes=False),
        scratch_types=(
            pltpu.VMEM((CH,), jnp.int32),
            pltpu.VMEM((CH,), jnp.int32),
            pltpu.VMEM((16 * LCAP,), jnp.int32),
            pltpu.VMEM((16,), jnp.int32),
            pltpu.VMEM((16 * SEGW,), jnp.float32),
            pltpu.VMEM((SEGW,), jnp.float32),
        ),
    )(_sc_preproc_body)


def _sc_preproc(ei_ui, ei_iu):
    return _build_preproc()(ei_ui[0], ei_ui[1], ei_iu[0], ei_iu[1])


def _sc_preproc_body(src_ui, dst_ui, src_iu, dst_iu,
                     bkt_ui, fil_ui, cnt_ui, bkt_iu, fil_iu,
                     cnt_iu, sbuf, dbuf, lst, fills, hist, comb):
    wid = _wid()
    lo = wid * T
    lane = _lane()

    for esrc, edst, bkt, fil, cnt in (
        (src_ui, dst_ui, bkt_ui, fil_ui, cnt_ui),
        (src_iu, dst_iu, bkt_iu, fil_iu, cnt_iu),
    ):
        def zh(i, _):
            hist[pl.ds(i * 16, 16)] = jnp.zeros((16,), jnp.float32)
            return 0

        lax.fori_loop(0, 16 * SEGW // 16, zh, 0)
        fills[...] = jnp.zeros((16,), jnp.int32)

        def chunk_body(c, _, esrc=esrc, edst=edst):
            pltpu.sync_copy(esrc.at[pl.ds(c * CH, CH)], sbuf)
            pltpu.sync_copy(edst.at[pl.ds(c * CH, CH)], dbuf)

            def vbody(g, _):
                s = sbuf[pl.ds(g * 16, 16)]
                dd = dbuf[pl.ds(g * 16, 16)]
                dl = dd - lo
                inr = (dl >= 0) & (dl < T)
                dlc = jnp.where(inr, jnp.clip(dl, 0, T), GARB)
                hk = lane * SEGW + dlc
                cur = plsc.load_gather(hist, [hk])
                plsc.store_scatter(hist, [hk],
                                   cur + jnp.where(inr, 1.0, 0.0))
                fl = fills[...]
                slot = jnp.where(inr, jnp.minimum(fl, LCAP - 2), LCAP - 1)
                wk = lane * LCAP + slot
                packed = s * 512 + jnp.clip(dl, 0, 511)
                plsc.store_scatter(lst, [wk], packed)
                fills[...] = fl + inr.astype(jnp.int32)
                return 0

            lax.fori_loop(0, CH // 16, vbody, 0)
            return 0

        lax.fori_loop(0, NCH, chunk_body, 0)

        pltpu.sync_copy(lst, bkt.at[wid])
        pltpu.sync_copy(fills, fil.at[wid])
        def ch_comb(i, _):
            acc = hist[pl.ds(i * 16, 16)]
            for l in range(1, 16):
                acc = acc + hist[pl.ds(l * SEGW + i * 16, 16)]
            comb[pl.ds(i * 16, 16)] = acc
            return 0

        lax.fori_loop(0, T // 16, ch_comb, 0)
        pltpu.sync_copy(comb.at[pl.ds(0, T)], cnt.at[pl.ds(lo, T)])


@functools.cache
def _build_edge():
    return functools.partial(
        pl.kernel,
        out_type=(
            jax.ShapeDtypeStruct((NP * D,), jnp.float32),
            jax.ShapeDtypeStruct((NP,), jnp.float32),
            jax.ShapeDtypeStruct((NP * D,), jnp.float32),
            jax.ShapeDtypeStruct((NP,), jnp.float32),
        ),
        mesh=_mesh(),
        compiler_params=pltpu.CompilerParams(needs_layout_passes=False),
        scratch_types=(
            pltpu.VMEM((16 * LCAP,), jnp.int32),
            pltpu.VMEM((16,), jnp.int32),
            pltpu.VMEM((N,), jnp.float32),
            pltpu.VMEM((16 * SEGW,), jnp.float32),
            pltpu.VMEM((SEGW,), jnp.float32),
            pltpu.VMEM((SEGW,), jnp.float32),
            pltpu.VMEM((SEGW,), jnp.float32),
            pltpu.VMEM((T * D,), jnp.float32),
            pltpu.VMEM((16,), jnp.int32),
            pltpu.VMEM((16,), jnp.int32),
            pltpu.VMEM((16, D), jnp.float32),
            pltpu.VMEM((16, D), jnp.float32),
            pltpu.VMEM((32,), jnp.float32),
            pltpu.VMEM((32,), jnp.int32),
            pltpu.VMEM((32,), jnp.float32),
            pltpu.VMEM((32,), jnp.int32),
            pltpu.SemaphoreType.DMA,
            pltpu.SemaphoreType.DMA,
        ),
    )(_sc_edge_body)


def _sc_edge(*args):
    return _build_edge()(*args)


def _sc_edge_body(p_ui, v_u, bkt_ui, fil_ui, cnt_ui,
                  p_iu, v_i, bkt_iu, fil_iu, cnt_iu,
                  numer_ui, s_ui, numer_iu, s_iu,
                  lst, fills, pbuf, stat, mcomb, scomb, degb, acc,
                  idx0, idx1, row0, row1, cfs0, dls0, cfs1, dls1,
                  sem0, sem1):
    wid = _wid()
    lo = wid * T
    lane = _lane()
    NEG = jnp.float32(-3.0e38)

    for p_in, v_in, bkt, fil, cnt, numer, s_out in (
        (p_ui, v_u, bkt_ui, fil_ui, cnt_ui, numer_ui, s_ui),
        (p_iu, v_i, bkt_iu, fil_iu, cnt_iu, numer_iu, s_iu),
    ):
        pltpu.sync_copy(bkt.at[wid], lst)
        pltpu.sync_copy(fil.at[wid], fills)
        pltpu.sync_copy(p_in, pbuf)
        pltpu.sync_copy(cnt.at[pl.ds(lo, T)], degb.at[pl.ds(0, T)])

        fl = fills[...]
        gmax = jnp.max(fl, axis=0)

        def unpack(g):
            pk = plsc.load_gather(lst, [lane * LCAP + g])
            valid = g < fills[...]
            dl = pk & 511
            sv = jnp.clip(lax.shift_right_logical(pk, 9), 0, N - 1)
            dlc = jnp.where(valid, jnp.clip(dl, 0, T), GARB)
            raw = plsc.load_gather(pbuf, [sv])
            return valid, dlc, sv, raw

        def zm(i, _):
            stat[pl.ds(i * 16, 16)] = jnp.full((16,), NEG, jnp.float32)
            return 0

        lax.fori_loop(0, 16 * SEGW // 16, zm, 0)

        def p1(g, _):
            valid, dlc, _, raw = unpack(g)
            hk = lane * SEGW + dlc
            cur = plsc.load_gather(stat, [hk])
            plsc.store_scatter(stat, [hk],
                               jnp.maximum(cur, jnp.where(valid, raw, NEG)))
            return 0

        lax.fori_loop(0, gmax, p1, 0)

        def mcmb(i, _):
            acc_v = stat[pl.ds(i * 16, 16)]
            for l in range(1, 16):
                acc_v = jnp.maximum(acc_v, stat[pl.ds(l * SEGW + i * 16, 16)])
            mcomb[pl.ds(i * 16, 16)] = acc_v
            return 0

        lax.fori_loop(0, SEGW // 16, mcmb, 0)

        def zs(i, _):
            stat[pl.ds(i * 16, 16)] = jnp.zeros((16,), jnp.float32)
            return 0

        lax.fori_loop(0, 16 * SEGW // 16, zs, 0)

        def p2(g, _):
            valid, dlc, _, raw = unpack(g)
            mg = plsc.load_gather(mcomb, [dlc])
            w = jnp.where(valid, jnp.exp(raw - mg), 0.0)
            hk = lane * SEGW + dlc
            cur = plsc.load_gather(stat, [hk])
            plsc.store_scatter(stat, [hk], cur + w)
            return 0

        lax.fori_loop(0, gmax, p2, 0)

        def scmb(i, _):
            acc_v = stat[pl.ds(i * 16, 16)]
            for l in range(1, 16):
                acc_v = acc_v + stat[pl.ds(l * SEGW + i * 16, 16)]
            scomb[pl.ds(i * 16, 16)] = acc_v
            return 0

        lax.fori_loop(0, SEGW // 16, scmb, 0)

        def za(i, _):
            acc[pl.ds(i * 16, 16)] = jnp.zeros((16,), jnp.float32)
            return 0

        lax.fori_loop(0, T * D // 16, za, 0)

        def stage(g, which):
            valid, dlc, sv, raw = unpack(g)
            mg = plsc.load_gather(mcomb, [dlc])
            w = jnp.exp(raw - mg)
            sg = plsc.load_gather(scomb, [dlc])
            dg = plsc.load_gather(degb, [dlc])
            coef = w / (jnp.maximum(sg, 0.5) * jnp.maximum(dg, 1.0))
            coef = jnp.where(valid, coef, 0.0)
            dsan = jnp.where(valid, jnp.clip(dlc, 0, T - 1), 0)
            idxr = idx0 if which == 0 else idx1
            idxr[...] = jnp.where(valid, sv, 0)
            (cfs0 if which == 0 else cfs1)[pl.ds(0, 16)] = coef
            (dls0 if which == 0 else dls1)[pl.ds(0, 16)] = dsan

        def start_copy(which, v_in=v_in):
            if which == 0:
                pltpu.async_copy(v_in.at[idx0], row0, sem0)
            else:
                pltpu.async_copy(v_in.at[idx1], row1, sem1)

        def wait_copy(which, v_in=v_in):
            if which == 0:
                pltpu.make_async_copy(v_in.at[idx0], row0, sem0).wait()
            else:
                pltpu.make_async_copy(v_in.at[idx1], row1, sem1).wait()

        @pl.when(gmax > 0)
        def _():
            stage(0, 0)
            start_copy(0)

            def b_body(b, _):
                even = lax.rem(b, 2)

                def process(rowbuf, cfs, dls):
                    def jbody(j, _):
                        c = cfs[pl.ds(j, 16)][0]
                        dd = dls[pl.ds(j, 16)][0]
                        off = dd * D
                        cvec = jnp.full((16,), c, jnp.float32)
                        for ch in range(16):
                            sl = pl.ds(off + ch * 16, 16)
                            rr = rowbuf[j, pl.ds(ch * 16, 16)]
                            acc[sl] = acc[sl] + cvec * rr
                        return 0

                    lax.fori_loop(0, 16, jbody, 0)

                @pl.when(even == 0)
                def _():
                    wait_copy(0)

                    @pl.when(b + 1 < gmax)
                    def _():
                        stage(b + 1, 1)
                        start_copy(1)

                    process(row0, cfs0, dls0)

                @pl.when(even == 1)
                def _():
                    wait_copy(1)

                    @pl.when(b + 1 < gmax)
                    def _():
                        stage(b + 1, 0)
                        start_copy(0)

                    process(row1, cfs1, dls1)

                return 0

            lax.fori_loop(0, gmax, b_body, 0)

        pltpu.sync_copy(acc, numer.at[pl.ds(lo * D, T * D)])
        pltpu.sync_copy(scomb.at[pl.ds(0, T)], s_out.at[pl.ds(lo, T)])


def _k1_body(x_ref, wcat_ref, bcat_ref, wet_ref, cnt_ref,
             qw_ref, v_ref, ra_ref, kbar_ref):
    i = pl.program_id(0)
    x = x_ref[...]
    q = jnp.dot(x, wcat_ref[:, 0:D], preferred_element_type=jnp.float32)
    q = q + bcat_ref[:, 0:D]
    k = jnp.dot(x, wcat_ref[:, D:2 * D], preferred_element_type=jnp.float32)
    k = k + bcat_ref[:, D:2 * D]
    v = jnp.dot(x, wcat_ref[:, 2 * D:3 * D], preferred_element_type=jnp.float32)
    v = v + bcat_ref[:, 2 * D:3 * D]
    ra_ref[...] = jnp.sum(q * k, axis=1, keepdims=True) * RSQRT_D
    v_ref[...] = v
    qw_ref[...] = jnp.dot(q, wet_ref[...], preferred_element_type=jnp.float32)
    part = jnp.sum(cnt_ref[...] * k, axis=0, keepdims=True)

    @pl.when(i == 0)
    def _():
        kbar_ref[...] = part

    @pl.when(i != 0)
    def _():
        kbar_ref[...] = kbar_ref[...] + part


def _k1(x, wcat, bcat, wet, cnt):
    return pl.pallas_call(
        _k1_body,
        grid=(GRID,),
        in_specs=[
            pl.BlockSpec((BLK, D), lambda i: (i, 0)),
            pl.BlockSpec((D, 3 * D), lambda i: (0, 0)),
            pl.BlockSpec((1, 3 * D), lambda i: (0, 0)),
            pl.BlockSpec((D, D), lambda i: (0, 0)),
            pl.BlockSpec((BLK, 1), lambda i: (i, 0)),
        ],
        out_specs=[
            pl.BlockSpec((BLK, D), lambda i: (i, 0)),
            pl.BlockSpec((BLK, D), lambda i: (i, 0)),
            pl.BlockSpec((BLK, 1), lambda i: (i, 0)),
            pl.BlockSpec((1, D), lambda i: (0, 0)),
        ],
        out_shape=[
            jax.ShapeDtypeStruct((N, D), jnp.float32),
            jax.ShapeDtypeStruct((N, D), jnp.float32),
            jax.ShapeDtypeStruct((N, 1), jnp.float32),
            jax.ShapeDtypeStruct((1, D), jnp.float32),
        ],
    )(x, wcat, bcat, wet, cnt)


def _k2_body(qwu_ref, kbi_ref, qwi_ref, kbu_ref, pui_ref, piu_ref):
    pui_ref[...] = jnp.sum(
        qwu_ref[...] * kbi_ref[...], axis=1, keepdims=True
    ) * RSQRT_D
    piu_ref[...] = jnp.sum(
        qwi_ref[...] * kbu_ref[...], axis=1, keepdims=True
    ) * RSQRT_D


def _k2(qw_u, kbar_i, qw_i, kbar_u):
    return pl.pallas_call(
        _k2_body,
        grid=(GRID,),
        in_specs=[
            pl.BlockSpec((BLK, D), lambda i: (i, 0)),
            pl.BlockSpec((1, D), lambda i: (0, 0)),
            pl.BlockSpec((BLK, D), lambda i: (i, 0)),
            pl.BlockSpec((1, D), lambda i: (0, 0)),
        ],
        out_specs=[
            pl.BlockSpec((BLK, 1), lambda i: (i, 0)),
            pl.BlockSpec((BLK, 1), lambda i: (i, 0)),
        ],
        out_shape=[
            jax.ShapeDtypeStruct((N, 1), jnp.float32),
            jax.ShapeDtypeStruct((N, 1), jnp.float32),
        ],
    )(qw_u, kbar_i, qw_i, kbar_u)


def _ksoft_body(rau_ref, rai_ref, mu_ref, zu_ref, mi_ref, zi_ref):
    for ra_ref, m_ref, z_ref in ((rau_ref, mu_ref, zu_ref),
                                 (rai_ref, mi_ref, zi_ref)):
        ra = ra_ref[...]
        m = jnp.max(ra)
        z = jnp.sum(jnp.exp(ra - m))
        m_ref[...] = jnp.full((1, 1), m, jnp.float32)
        z_ref[...] = jnp.full((1, 1), z, jnp.float32)


def _ksoft(ra_u, ra_i):
    return pl.pallas_call(
        _ksoft_body,
        grid=(1,),
        in_specs=[
            pl.BlockSpec((N, 1), lambda i: (0, 0)),
            pl.BlockSpec((N, 1), lambda i: (0, 0)),
        ],
        out_specs=[pl.BlockSpec((1, 1), lambda i: (0, 0))] * 4,
        out_shape=[jax.ShapeDtypeStruct((1, 1), jnp.float32)] * 4,
    )(ra_u, ra_i)


def _kpost_body(num_ref, ra_ref, v_ref, m_ref, z_ref, o_ref):
    alpha = jnp.exp(ra_ref[...] - m_ref[0, 0]) * (1.0 / z_ref[0, 0])
    o_ref[...] = alpha * v_ref[...] + num_ref[...]


def _kpost(numer, ra, v, m, z):
    return pl.pallas_call(
        _kpost_body,
        grid=(GRID,),
        in_specs=[
            pl.BlockSpec((BLK, D), lambda i: (i, 0)),
            pl.BlockSpec((BLK, 1), lambda i: (i, 0)),
            pl.BlockSpec((BLK, D), lambda i: (i, 0)),
            pl.BlockSpec((1, 1), lambda i: (0, 0)),
            pl.BlockSpec((1, 1), lambda i: (0, 0)),
        ],
        out_specs=pl.BlockSpec((BLK, D), lambda i: (i, 0)),
        out_shape=jax.ShapeDtypeStruct((N, D), jnp.float32),
    )(numer, ra, v, m, z)


def _kfin_body(x_ref, w_ref, b_ref, o_ref):
    o_ref[...] = x_ref[...] + jnp.dot(
        x_ref[...], w_ref[...], preferred_element_type=jnp.float32
    ) + b_ref[...]


def _kfin(x, w, b):
    return pl.pallas_call(
        _kfin_body,
        grid=(GRID,),
        in_specs=[
            pl.BlockSpec((BLK, D), lambda i: (i, 0)),
            pl.BlockSpec((D, D), lambda i: (0, 0)),
            pl.BlockSpec((1, D), lambda i: (0, 0)),
        ],
        out_specs=pl.BlockSpec((BLK, D), lambda i: (i, 0)),
        out_shape=jax.ShapeDtypeStruct((N, D), jnp.float32),
    )(x, w.T, b[None, :])


def _mm_kernel(x_ref, w_ref, b_ref, o_ref):
    o_ref[...] = (
        jnp.dot(x_ref[...], w_ref[...], preferred_element_type=jnp.float32)
        + b_ref[...]
    )


def _lin_pallas(x, w, b):
    return pl.pallas_call(
        _mm_kernel,
        grid=(GRID,),
        in_specs=[
            pl.BlockSpec((BLK, D), lambda i: (i, 0)),
            pl.BlockSpec((D, D), lambda i: (0, 0)),
            pl.BlockSpec((1, D), lambda i: (0, 0)),
        ],
        out_specs=pl.BlockSpec((BLK, D), lambda i: (i, 0)),
        out_shape=jax.ShapeDtypeStruct((N, D), jnp.float32),
    )(x, w.T, b[None, :])


_ZB = None


def _layer(xu, xi, lp, bkt_ui, fil_ui, cnt_ui, bkt_iu, fil_iu, cnt_iu):
    zb = jnp.zeros((D,), jnp.float32)
    q_u = _lin_pallas(xu, lp["q"]["user"]["w"], lp["q"]["user"]["b"])
    k_u = _lin_pallas(xu, lp["k"]["user"]["w"], lp["k"]["user"]["b"])
    v_u = _lin_pallas(xu, lp["v"]["user"]["w"], lp["v"]["user"]["b"])
    q_i = _lin_pallas(xi, lp["q"]["item"]["w"], lp["q"]["item"]["b"])
    k_i = _lin_pallas(xi, lp["k"]["item"]["w"], lp["k"]["item"]["b"])
    v_i = _lin_pallas(xi, lp["v"]["item"]["w"], lp["v"]["item"]["b"])
    qw_u = _lin_pallas(q_u, lp["W"]["user__item"].T, zb)
    qw_i = _lin_pallas(q_i, lp["W"]["item__user"].T, zb)

    kbar_i = jnp.sum(cnt_ui[:N, None] * k_i, axis=0)
    kbar_u = jnp.sum(cnt_iu[:N, None] * k_u, axis=0)
    p_ui = (qw_u @ kbar_i) * RSQRT_D
    p_iu = (qw_i @ kbar_u) * RSQRT_D

    numer_ui, s_ui, numer_iu, s_iu = _sc_edge(
        p_ui, v_u, bkt_ui, fil_ui, cnt_ui,
        p_iu, v_i, bkt_iu, fil_iu, cnt_iu,
    )
    del s_ui, s_iu

    ra_u = jnp.sum(q_u * k_u, axis=-1) * RSQRT_D
    ra_i = jnp.sum(q_i * k_i, axis=-1) * RSQRT_D
    self_u = jax.nn.softmax(ra_u, axis=0)[:, None] * v_u
    self_i = jax.nn.softmax(ra_i, axis=0)[:, None] * v_i

    out_i = jnp.reshape(numer_ui, (NP, D))[:N] + self_i
    out_u = jnp.reshape(numer_iu, (NP, D))[:N] + self_u
    return out_u, out_i


def kernel(x_user, x_item, edge_index_user_to_item, edge_index_item_to_user,
           params):
    ei_ui = edge_index_user_to_item.astype(jnp.int32)
    ei_iu = edge_index_item_to_user.astype(jnp.int32)
    bkt_ui, fil_ui, cnt_ui, bkt_iu, fil_iu, cnt_iu = _sc_preproc(ei_ui, ei_iu)

    xu, xi = x_user, x_item
    for lname in ("l1", "l2"):
        xu, xi = _layer(
            xu, xi, params[lname],
            bkt_ui, fil_ui, cnt_ui, bkt_iu, fil_iu, cnt_iu,
        )
    yu = _kfin(xu, params["self"]["user"]["w"], params["self"]["user"]["b"])
    yi = _kfin(xi, params["self"]["item"]["w"], params["self"]["item"]["b"])
    return (yu, yi)
